# baseline (speedup 1.0000x reference)
import functools

import jax
import jax.numpy as jnp
from jax.experimental import pallas as pl


def _chamfer_body(x1_ref, x2_ref, dist_ref, idx_ref, *, M):
    a = x1_ref[0]
    b = x2_ref[0]

    ax, ay, az = a[0:1, :], a[1:2, :], a[2:3, :]
    bx, by, bz = b[:, 0:1], b[:, 1:2], b[:, 2:3]

    x2 = ax * ax + ay * ay + az * az
    y2 = bx * bx + by * by + bz * bz

    f32 = jnp.float32
    axl, ayl, azl = (v.astype(jnp.bfloat16).astype(f32) for v in (ax, ay, az))
    bxl, byl, bzl = (v.astype(jnp.bfloat16).astype(f32) for v in (bx, by, bz))
    xy = bxl * axl + byl * ayl + bzl * azl
    d = (x2 + y2) - 2.0 * xy

    dmin = jnp.min(d, axis=0, keepdims=True)
    iota = jax.lax.broadcasted_iota(jnp.int32, d.shape, 0)
    masked = jnp.where(d == dmin, iota, M)
    imin = jnp.min(masked, axis=0, keepdims=True)

    dist_ref[0] = dmin
    idx_ref[0] = imin


@jax.jit
def kernel(xyz1, xyz2):
    xyz1 = xyz1.astype(jnp.float32)
    xyz2 = xyz2.astype(jnp.float32)
    B, N, _ = xyz1.shape
    M = xyz2.shape[1]

    TN = 512
    NB = N // TN

    x1t = jnp.transpose(xyz1, (0, 2, 1))

    grid = (B * NB,)
    dist, idx = pl.pallas_call(
        functools.partial(_chamfer_body, M=M),
        grid=grid,
        in_specs=[
            pl.BlockSpec((1, 3, TN), lambda g: (g // NB, 0, g % NB)),
            pl.BlockSpec((1, M, 3), lambda g: (g // NB, 0, 0)),
        ],
        out_specs=[
            pl.BlockSpec((1, 1, TN), lambda g: (g, 0, 0)),
            pl.BlockSpec((1, 1, TN), lambda g: (g, 0, 0)),
        ],
        out_shape=[
            jax.ShapeDtypeStruct((B * NB, 1, TN), jnp.float32),
            jax.ShapeDtypeStruct((B * NB, 1, TN), jnp.int32),
        ],
    )(x1t, xyz2)

    return dist.reshape(B, N), idx.reshape(B, N)


# speedup vs baseline: 1.4199x; 1.4199x over previous
import functools

import jax
import jax.numpy as jnp
from jax.experimental import pallas as pl
from jax.experimental.pallas import tpu as pltpu

_TN = 128
_MC = 64


def _chamfer_body(x1_ref, x2_ref, dist_ref, idx_ref,
                  bx_s, by_s, bz_s, y2_s, *, M, NB):
    f32 = jnp.float32
    step = pl.program_id(0)

    @pl.when(step % NB == 0)
    def _build_scratch():
        b = x2_ref[0]
        bx, by, bz = b[:, 0:1], b[:, 1:2], b[:, 2:3]
        y2 = bx * bx + by * by + bz * bz
        rnd = lambda v: v.astype(jnp.bfloat16).astype(f32) * -2.0
        bx_s[...] = jnp.broadcast_to(rnd(bx), (M, _TN))
        by_s[...] = jnp.broadcast_to(rnd(by), (M, _TN))
        bz_s[...] = jnp.broadcast_to(rnd(bz), (M, _TN))
        y2_s[...] = jnp.broadcast_to(y2, (M, _TN))

    a = x1_ref[0]
    ax, ay, az = a[0:1, :], a[1:2, :], a[2:3, :]
    x2 = ax * ax + ay * ay + az * az
    rnd_a = lambda v: v.astype(jnp.bfloat16).astype(f32)
    axl, ayl, azl = rnd_a(ax), rnd_a(ay), rnd_a(az)

    iota = jax.lax.broadcasted_iota(jnp.int32, (_MC, _TN), 0)

    runmin = jnp.full((_MC, _TN), jnp.inf, f32)
    runidx = jnp.zeros((_MC, _TN), jnp.int32)
    for c in range(M // _MC):
        off = c * _MC
        bx = bx_s[off:off + _MC, :]
        by = by_s[off:off + _MC, :]
        bz = bz_s[off:off + _MC, :]
        y2 = y2_s[off:off + _MC, :]
        u = bx * axl + by * ayl + bz * azl
        d = (x2 + y2) + u
        mask = d < runmin
        runmin = jnp.where(mask, d, runmin)
        runidx = jnp.where(mask, iota + off, runidx)

    dmin = jnp.min(runmin, axis=0, keepdims=True)
    cand = jnp.where(runmin == dmin, runidx, M)
    imin = jnp.min(cand, axis=0, keepdims=True)

    dist_ref[0] = dmin
    idx_ref[0] = imin


@jax.jit
def kernel(xyz1, xyz2):
    xyz1 = xyz1.astype(jnp.float32)
    xyz2 = xyz2.astype(jnp.float32)
    B, N, _ = xyz1.shape
    M = xyz2.shape[1]
    NB = N // _TN

    x1t = jnp.transpose(xyz1, (0, 2, 1))

    grid = (B * NB,)
    dist, idx = pl.pallas_call(
        functools.partial(_chamfer_body, M=M, NB=NB),
        grid=grid,
        in_specs=[
            pl.BlockSpec((1, 3, _TN), lambda g: (g // NB, 0, g % NB)),
            pl.BlockSpec((1, M, 3), lambda g: (g // NB, 0, 0)),
        ],
        out_specs=[
            pl.BlockSpec((1, 1, _TN), lambda g: (g, 0, 0)),
            pl.BlockSpec((1, 1, _TN), lambda g: (g, 0, 0)),
        ],
        out_shape=[
            jax.ShapeDtypeStruct((B * NB, 1, _TN), jnp.float32),
            jax.ShapeDtypeStruct((B * NB, 1, _TN), jnp.int32),
        ],
        scratch_shapes=[
            pltpu.VMEM((M, _TN), jnp.float32),
            pltpu.VMEM((M, _TN), jnp.float32),
            pltpu.VMEM((M, _TN), jnp.float32),
            pltpu.VMEM((M, _TN), jnp.float32),
        ],
    )(x1t, xyz2)

    return dist.reshape(B, N), idx.reshape(B, N)


# speedup vs baseline: 1.9759x; 1.3916x over previous
import functools

import jax
import jax.numpy as jnp
from jax.experimental import pallas as pl
from jax.experimental.pallas import tpu as pltpu

_TN = 128
_MC = 64
_K = 8


def _chamfer_body(x1_ref, x2_ref, dist_ref, idx_ref,
                  bneg_s, y2_s, *, M, NB):
    f32 = jnp.float32
    bf16 = jnp.bfloat16
    step = pl.program_id(0)

    @pl.when(step % NB == 0)
    def _build_scratch():
        b = x2_ref[0]
        bx, by, bz = b[:, 0:1], b[:, 1:2], b[:, 2:3]
        y2 = bx * bx + by * by + bz * bz
        bneg = b.astype(bf16) * jnp.asarray(-2.0, bf16)
        bneg_s[:, 0:3] = bneg
        bneg_s[:, 3:_K] = jnp.zeros((M, _K - 3), bf16)
        y2_s[...] = jnp.broadcast_to(y2, (M, _TN))

    a = x1_ref[0]
    ax, ay, az = a[0:1, :], a[1:2, :], a[2:3, :]
    x2 = ax * ax + ay * ay + az * az
    a8 = jnp.concatenate(
        [a.astype(bf16), jnp.zeros((_K - 3, _TN), bf16)], axis=0)

    iota = jax.lax.broadcasted_iota(jnp.int32, (_MC, _TN), 0)
    dims = (((1,), (0,)), ((), ()))

    runmin = jnp.full((_MC, _TN), jnp.inf, f32)
    runidx = jnp.zeros((_MC, _TN), jnp.int32)
    for c in range(M // _MC):
        off = c * _MC
        bneg = bneg_s[off:off + _MC, :]
        y2 = y2_s[off:off + _MC, :]
        u = jax.lax.dot_general(bneg, a8, dims,
                                preferred_element_type=f32)
        d = (x2 + y2) + u
        mask = d < runmin
        runmin = jnp.where(mask, d, runmin)
        runidx = jnp.where(mask, iota + off, runidx)

    dmin = jnp.min(runmin, axis=0, keepdims=True)
    cand = jnp.where(runmin == dmin, runidx, M)
    imin = jnp.min(cand, axis=0, keepdims=True)

    dist_ref[0] = dmin
    idx_ref[0] = imin


@jax.jit
def kernel(xyz1, xyz2):
    xyz1 = xyz1.astype(jnp.float32)
    xyz2 = xyz2.astype(jnp.float32)
    B, N, _ = xyz1.shape
    M = xyz2.shape[1]
    NB = N // _TN

    x1t = jnp.transpose(xyz1, (0, 2, 1))

    grid = (B * NB,)
    dist, idx = pl.pallas_call(
        functools.partial(_chamfer_body, M=M, NB=NB),
        grid=grid,
        in_specs=[
            pl.BlockSpec((1, 3, _TN), lambda g: (g // NB, 0, g % NB)),
            pl.BlockSpec((1, M, 3), lambda g: (g // NB, 0, 0)),
        ],
        out_specs=[
            pl.BlockSpec((1, 1, _TN), lambda g: (g, 0, 0)),
            pl.BlockSpec((1, 1, _TN), lambda g: (g, 0, 0)),
        ],
        out_shape=[
            jax.ShapeDtypeStruct((B * NB, 1, _TN), jnp.float32),
            jax.ShapeDtypeStruct((B * NB, 1, _TN), jnp.int32),
        ],
        scratch_shapes=[
            pltpu.VMEM((M, _K), jnp.bfloat16),
            pltpu.VMEM((M, _TN), jnp.float32),
        ],
    )(x1t, xyz2)

    return dist.reshape(B, N), idx.reshape(B, N)


# speedup vs baseline: 2.5153x; 1.2730x over previous
import functools

import jax
import jax.numpy as jnp
from jax.experimental import pallas as pl
from jax.experimental.pallas import tpu as pltpu

_TN = 256
_MC = 64
_K = 8


def _chamfer_body(x1_ref, x2_ref, dist_ref, idx_ref,
                  bneg_s, y2_s, *, M, NB):
    f32 = jnp.float32
    bf16 = jnp.bfloat16
    step = pl.program_id(0)

    @pl.when(step % NB == 0)
    def _build_scratch():
        b = x2_ref[0]
        bx, by, bz = b[:, 0:1], b[:, 1:2], b[:, 2:3]
        y2 = bx * bx + by * by + bz * bz
        bneg = b.astype(bf16) * jnp.asarray(-2.0, bf16)
        bneg_s[:, 0:3] = bneg
        bneg_s[:, 3:_K] = jnp.zeros((M, _K - 3), bf16)
        y2_s[...] = jnp.broadcast_to(y2, (M, _TN))

    a = x1_ref[0]
    ax, ay, az = a[0:1, :], a[1:2, :], a[2:3, :]
    x2 = ax * ax + ay * ay + az * az
    a8 = jnp.concatenate(
        [a.astype(bf16), jnp.zeros((_K - 3, _TN), bf16)], axis=0)

    iota = jax.lax.broadcasted_iota(jnp.int32, (_MC, _TN), 0)
    dims = (((1,), (0,)), ((), ()))

    runmin = jnp.full((_MC, _TN), jnp.inf, f32)
    runidx = jnp.zeros((_MC, _TN), jnp.int32)
    for c in range(M // _MC):
        off = c * _MC
        bneg = bneg_s[off:off + _MC, :]
        y2 = y2_s[off:off + _MC, :]
        u = jax.lax.dot_general(bneg, a8, dims,
                                preferred_element_type=f32)
        d = (x2 + y2) + u
        mask = d < runmin
        runmin = jnp.where(mask, d, runmin)
        runidx = jnp.where(mask, iota + off, runidx)

    dmin = jnp.min(runmin, axis=0, keepdims=True)
    cand = jnp.where(runmin == dmin, runidx, M)
    imin = jnp.min(cand, axis=0, keepdims=True)

    dist_ref[0] = dmin
    idx_ref[0] = imin


@jax.jit
def kernel(xyz1, xyz2):
    xyz1 = xyz1.astype(jnp.float32)
    xyz2 = xyz2.astype(jnp.float32)
    B, N, _ = xyz1.shape
    M = xyz2.shape[1]
    NB = N // _TN

    x1t = jnp.transpose(xyz1, (0, 2, 1))

    grid = (B * NB,)
    dist, idx = pl.pallas_call(
        functools.partial(_chamfer_body, M=M, NB=NB),
        grid=grid,
        in_specs=[
            pl.BlockSpec((1, 3, _TN), lambda g: (g // NB, 0, g % NB)),
            pl.BlockSpec((1, M, 3), lambda g: (g // NB, 0, 0)),
        ],
        out_specs=[
            pl.BlockSpec((1, 1, _TN), lambda g: (g, 0, 0)),
            pl.BlockSpec((1, 1, _TN), lambda g: (g, 0, 0)),
        ],
        out_shape=[
            jax.ShapeDtypeStruct((B * NB, 1, _TN), jnp.float32),
            jax.ShapeDtypeStruct((B * NB, 1, _TN), jnp.int32),
        ],
        scratch_shapes=[
            pltpu.VMEM((M, _K), jnp.bfloat16),
            pltpu.VMEM((M, _TN), jnp.float32),
        ],
    )(x1t, xyz2)

    return dist.reshape(B, N), idx.reshape(B, N)


# speedup vs baseline: 2.5794x; 1.0255x over previous
import functools

import jax
import jax.numpy as jnp
from jax.experimental import pallas as pl
from jax.experimental.pallas import tpu as pltpu

_TN = 512
_MC = 64
_K = 8


def _chamfer_body(x1_ref, x2_ref, dist_ref, idx_ref,
                  bneg_s, y2_s, *, M, NB):
    f32 = jnp.float32
    bf16 = jnp.bfloat16
    step = pl.program_id(0)

    @pl.when(step % NB == 0)
    def _build_scratch():
        b = x2_ref[0]
        bx, by, bz = b[:, 0:1], b[:, 1:2], b[:, 2:3]
        y2 = bx * bx + by * by + bz * bz
        bneg = b.astype(bf16) * jnp.asarray(-2.0, bf16)
        bneg_s[:, 0:3] = bneg
        bneg_s[:, 3:_K] = jnp.zeros((M, _K - 3), bf16)
        y2_s[...] = jnp.broadcast_to(y2, (M, _TN))

    a = x1_ref[0]
    ax, ay, az = a[0:1, :], a[1:2, :], a[2:3, :]
    x2 = ax * ax + ay * ay + az * az
    a8 = jnp.concatenate(
        [a.astype(bf16), jnp.zeros((_K - 3, _TN), bf16)], axis=0)

    iota = jax.lax.broadcasted_iota(jnp.int32, (_MC, _TN), 0)
    dims = (((1,), (0,)), ((), ()))

    runmin = jnp.full((_MC, _TN), jnp.inf, f32)
    runidx = jnp.zeros((_MC, _TN), jnp.int32)
    for c in range(M // _MC):
        off = c * _MC
        bneg = bneg_s[off:off + _MC, :]
        y2 = y2_s[off:off + _MC, :]
        u = jax.lax.dot_general(bneg, a8, dims,
                                preferred_element_type=f32)
        d = (x2 + y2) + u
        mask = d < runmin
        runmin = jnp.where(mask, d, runmin)
        runidx = jnp.where(mask, iota + off, runidx)

    dmin = jnp.min(runmin, axis=0, keepdims=True)
    cand = jnp.where(runmin == dmin, runidx, M)
    imin = jnp.min(cand, axis=0, keepdims=True)

    dist_ref[0] = dmin
    idx_ref[0] = imin


@jax.jit
def kernel(xyz1, xyz2):
    xyz1 = xyz1.astype(jnp.float32)
    xyz2 = xyz2.astype(jnp.float32)
    B, N, _ = xyz1.shape
    M = xyz2.shape[1]
    NB = N // _TN

    x1t = jnp.transpose(xyz1, (0, 2, 1))

    grid = (B * NB,)
    dist, idx = pl.pallas_call(
        functools.partial(_chamfer_body, M=M, NB=NB),
        grid=grid,
        in_specs=[
            pl.BlockSpec((1, 3, _TN), lambda g: (g // NB, 0, g % NB)),
            pl.BlockSpec((1, M, 3), lambda g: (g // NB, 0, 0)),
        ],
        out_specs=[
            pl.BlockSpec((1, 1, _TN), lambda g: (g, 0, 0)),
            pl.BlockSpec((1, 1, _TN), lambda g: (g, 0, 0)),
        ],
        out_shape=[
            jax.ShapeDtypeStruct((B * NB, 1, _TN), jnp.float32),
            jax.ShapeDtypeStruct((B * NB, 1, _TN), jnp.int32),
        ],
        scratch_shapes=[
            pltpu.VMEM((M, _K), jnp.bfloat16),
            pltpu.VMEM((M, _TN), jnp.float32),
        ],
    )(x1t, xyz2)

    return dist.reshape(B, N), idx.reshape(B, N)


# speedup vs baseline: 3.2162x; 1.2469x over previous
import functools

import jax
import jax.numpy as jnp
from jax.experimental import pallas as pl
from jax.experimental.pallas import tpu as pltpu

_TN = 512
_MC = 64
_K = 8


def _chamfer_body(x1_ref, x2_ref, dist_ref, idx_ref,
                  bneg_s, *, M, NB):
    f32 = jnp.float32
    bf16 = jnp.bfloat16
    step = pl.program_id(0)

    @pl.when(step % NB == 0)
    def _build_scratch():
        b = x2_ref[0]
        bx, by, bz = b[:, 0:1], b[:, 1:2], b[:, 2:3]
        y2 = bx * bx + by * by + bz * bz
        y2a = y2.astype(bf16)
        r1 = y2 - y2a.astype(f32)
        y2b = r1.astype(bf16)
        r2 = r1 - y2b.astype(f32)
        y2c = r2.astype(bf16)
        bneg_s[:, 0:3] = b.astype(bf16) * jnp.asarray(-2.0, bf16)
        bneg_s[:, 3:4] = y2a
        bneg_s[:, 4:5] = y2b
        bneg_s[:, 5:6] = y2c
        bneg_s[:, 6:_K] = jnp.ones((M, _K - 6), bf16)

    a = x1_ref[0]
    ax, ay, az = a[0:1, :], a[1:2, :], a[2:3, :]
    x2 = ax * ax + ay * ay + az * az
    x2a = x2.astype(bf16)
    x2b = (x2 - x2a.astype(f32)).astype(bf16)
    ones = jnp.ones((_K - 5, _TN), bf16)
    a8 = jnp.concatenate([a.astype(bf16), ones, x2a, x2b], axis=0)

    dims = (((1,), (0,)), ((), ()))

    runmin = jnp.full((_MC, _TN), jnp.inf, f32)
    runc = jnp.zeros((_MC, _TN), jnp.int32)
    for c in range(M // _MC):
        bneg = bneg_s[c * _MC:(c + 1) * _MC, :]
        d = jax.lax.dot_general(bneg, a8, dims,
                                preferred_element_type=f32)
        mask = d < runmin
        runmin = jnp.where(mask, d, runmin)
        runc = jnp.where(mask, c, runc)

    dmin = jnp.min(runmin, axis=0, keepdims=True)
    rowiota = jax.lax.broadcasted_iota(jnp.int32, (_MC, _TN), 0)
    cand = jnp.where(runmin == dmin, runc * _MC + rowiota, M)
    imin = jnp.min(cand, axis=0, keepdims=True)

    dist_ref[0] = dmin
    idx_ref[0] = imin


@jax.jit
def kernel(xyz1, xyz2):
    xyz1 = xyz1.astype(jnp.float32)
    xyz2 = xyz2.astype(jnp.float32)
    B, N, _ = xyz1.shape
    M = xyz2.shape[1]
    NB = N // _TN

    x1t = jnp.transpose(xyz1, (0, 2, 1))

    grid = (B * NB,)
    dist, idx = pl.pallas_call(
        functools.partial(_chamfer_body, M=M, NB=NB),
        grid=grid,
        in_specs=[
            pl.BlockSpec((1, 3, _TN), lambda g: (g // NB, 0, g % NB)),
            pl.BlockSpec((1, M, 3), lambda g: (g // NB, 0, 0)),
        ],
        out_specs=[
            pl.BlockSpec((1, 1, _TN), lambda g: (g, 0, 0)),
            pl.BlockSpec((1, 1, _TN), lambda g: (g, 0, 0)),
        ],
        out_shape=[
            jax.ShapeDtypeStruct((B * NB, 1, _TN), jnp.float32),
            jax.ShapeDtypeStruct((B * NB, 1, _TN), jnp.int32),
        ],
        scratch_shapes=[
            pltpu.VMEM((M, _K), jnp.bfloat16),
        ],
    )(x1t, xyz2)

    return dist.reshape(B, N), idx.reshape(B, N)


# speedup vs baseline: 3.4507x; 1.0729x over previous
import functools

import jax
import jax.numpy as jnp
from jax.experimental import pallas as pl
from jax.experimental.pallas import tpu as pltpu

_TN = 256
_MC = 128
_K = 8


def _chamfer_body(x1_ref, x2_ref, dist_ref, idx_ref,
                  bneg_s, *, M, NB):
    f32 = jnp.float32
    bf16 = jnp.bfloat16
    step = pl.program_id(0)

    @pl.when(step % NB == 0)
    def _build_scratch():
        b = x2_ref[0]
        bx, by, bz = b[0:1, :], b[1:2, :], b[2:3, :]
        y2 = bx * bx + by * by + bz * bz
        y2a = y2.astype(bf16)
        r1 = y2 - y2a.astype(f32)
        y2b = r1.astype(bf16)
        r2 = r1 - y2b.astype(f32)
        y2c = r2.astype(bf16)
        bneg_s[0:3, :] = b.astype(bf16) * jnp.asarray(-2.0, bf16)
        bneg_s[3:4, :] = y2a
        bneg_s[4:5, :] = y2b
        bneg_s[5:6, :] = y2c
        bneg_s[6:_K, :] = jnp.ones((_K - 6, M), bf16)

    a = x1_ref[0]
    ax, ay, az = a[0:1, :], a[1:2, :], a[2:3, :]
    x2 = ax * ax + ay * ay + az * az
    x2a = x2.astype(bf16)
    x2b = (x2 - x2a.astype(f32)).astype(bf16)
    ones = jnp.ones((_K - 5, _TN), bf16)
    a8 = jnp.concatenate([a.astype(bf16), ones, x2a, x2b], axis=0)

    dims = (((0,), (0,)), ((), ()))

    runmin = jnp.full((_MC, _TN), jnp.inf, f32)
    runc = jnp.zeros((_MC, _TN), jnp.int32)
    for c in range(M // _MC):
        bneg = bneg_s[:, c * _MC:(c + 1) * _MC]
        d = jax.lax.dot_general(bneg, a8, dims,
                                preferred_element_type=f32)
        mask = d < runmin
        runmin = jnp.where(mask, d, runmin)
        runc = jnp.where(mask, c, runc)

    dmin = jnp.min(runmin, axis=0, keepdims=True)
    rowiota = jax.lax.broadcasted_iota(jnp.int32, (_MC, _TN), 0)
    cand = jnp.where(runmin == dmin, runc * _MC + rowiota, M)
    imin = jnp.min(cand, axis=0, keepdims=True)

    dist_ref[0] = dmin
    idx_ref[0] = imin


@jax.jit
def kernel(xyz1, xyz2):
    xyz1 = xyz1.astype(jnp.float32)
    xyz2 = xyz2.astype(jnp.float32)
    B, N, _ = xyz1.shape
    M = xyz2.shape[1]
    NB = N // _TN

    x1t = jnp.transpose(xyz1, (0, 2, 1))
    x2t = jnp.transpose(xyz2, (0, 2, 1))

    grid = (B * NB,)
    dist, idx = pl.pallas_call(
        functools.partial(_chamfer_body, M=M, NB=NB),
        grid=grid,
        in_specs=[
            pl.BlockSpec((1, 3, _TN), lambda g: (g // NB, 0, g % NB)),
            pl.BlockSpec((1, 3, M), lambda g: (g // NB, 0, 0)),
        ],
        out_specs=[
            pl.BlockSpec((1, 1, _TN), lambda g: (g, 0, 0)),
            pl.BlockSpec((1, 1, _TN), lambda g: (g, 0, 0)),
        ],
        out_shape=[
            jax.ShapeDtypeStruct((B * NB, 1, _TN), jnp.float32),
            jax.ShapeDtypeStruct((B * NB, 1, _TN), jnp.int32),
        ],
        scratch_shapes=[
            pltpu.VMEM((_K, M), jnp.bfloat16),
        ],
    )(x1t, x2t)

    return dist.reshape(B, N), idx.reshape(B, N)


# speedup vs baseline: 4.1293x; 1.1966x over previous
import functools

import jax
import jax.numpy as jnp
from jax.experimental import pallas as pl
from jax.experimental.pallas import tpu as pltpu

_TN = 512
_MC = 128
_K = 8


def _chamfer_body(x1_ref, x2_ref, dist_ref, idx_ref,
                  bneg_s, *, M, NB):
    f32 = jnp.float32
    bf16 = jnp.bfloat16
    step = pl.program_id(0)

    @pl.when(step % NB == 0)
    def _build_scratch():
        b = x2_ref[0]
        bx, by, bz = b[0:1, :], b[1:2, :], b[2:3, :]
        y2 = bx * bx + by * by + bz * bz
        y2a = y2.astype(bf16)
        r1 = y2 - y2a.astype(f32)
        y2b = r1.astype(bf16)
        r2 = r1 - y2b.astype(f32)
        y2c = r2.astype(bf16)
        bneg_s[0:3, :] = b.astype(bf16) * jnp.asarray(-2.0, bf16)
        bneg_s[3:4, :] = y2a
        bneg_s[4:5, :] = y2b
        bneg_s[5:6, :] = y2c
        bneg_s[6:_K, :] = jnp.ones((_K - 6, M), bf16)

    a = x1_ref[0]
    ax, ay, az = a[0:1, :], a[1:2, :], a[2:3, :]
    x2 = ax * ax + ay * ay + az * az
    x2a = x2.astype(bf16)
    x2b = (x2 - x2a.astype(f32)).astype(bf16)
    ones = jnp.ones((_K - 5, _TN), bf16)
    a8 = jnp.concatenate([a.astype(bf16), ones, x2a, x2b], axis=0)

    dims = (((0,), (0,)), ((), ()))

    runmin = jnp.full((_MC, _TN), jnp.inf, f32)
    runc = jnp.zeros((_MC, _TN), jnp.int32)
    for c in range(M // _MC):
        bneg = bneg_s[:, c * _MC:(c + 1) * _MC]
        d = jax.lax.dot_general(bneg, a8, dims,
                                preferred_element_type=f32)
        mask = d < runmin
        runmin = jnp.where(mask, d, runmin)
        runc = jnp.where(mask, c, runc)

    dmin = jnp.min(runmin, axis=0, keepdims=True)
    rowiota = jax.lax.broadcasted_iota(jnp.int32, (_MC, _TN), 0)
    cand = jnp.where(runmin == dmin, runc * _MC + rowiota, M)
    imin = jnp.min(cand, axis=0, keepdims=True)

    dist_ref[0] = dmin
    idx_ref[0] = imin


@jax.jit
def kernel(xyz1, xyz2):
    xyz1 = xyz1.astype(jnp.float32)
    xyz2 = xyz2.astype(jnp.float32)
    B, N, _ = xyz1.shape
    M = xyz2.shape[1]
    NB = N // _TN

    x1t = jnp.transpose(xyz1, (0, 2, 1))
    x2t = jnp.transpose(xyz2, (0, 2, 1))

    grid = (B * NB,)
    dist, idx = pl.pallas_call(
        functools.partial(_chamfer_body, M=M, NB=NB),
        grid=grid,
        in_specs=[
            pl.BlockSpec((1, 3, _TN), lambda g: (g // NB, 0, g % NB)),
            pl.BlockSpec((1, 3, M), lambda g: (g // NB, 0, 0)),
        ],
        out_specs=[
            pl.BlockSpec((1, 1, _TN), lambda g: (g, 0, 0)),
            pl.BlockSpec((1, 1, _TN), lambda g: (g, 0, 0)),
        ],
        out_shape=[
            jax.ShapeDtypeStruct((B * NB, 1, _TN), jnp.float32),
            jax.ShapeDtypeStruct((B * NB, 1, _TN), jnp.int32),
        ],
        scratch_shapes=[
            pltpu.VMEM((_K, M), jnp.bfloat16),
        ],
    )(x1t, x2t)

    return dist.reshape(B, N), idx.reshape(B, N)
